# Initial kernel scaffold; baseline (speedup 1.0000x reference)
import jax
import jax.numpy as jnp
from jax.experimental import pallas as pl


def kernel(x, position_embeddings):
    raise NotImplementedError("write your pallas kernel here")


# speedup vs baseline: 1.9229x; 1.9229x over previous
import math

import jax
import jax.numpy as jnp
from jax.experimental import pallas as pl


def _pe_add_kernel(x_ref, pe_ref, o_ref, *, scale):
    o_ref[...] = x_ref[...] * scale + pe_ref[...]


def kernel(x, position_embeddings):
    B, S, D = x.shape
    scale = math.sqrt(D)
    BLK = 512
    grid = (S // BLK, B)
    import functools

    return pl.pallas_call(
        functools.partial(_pe_add_kernel, scale=scale),
        grid=grid,
        in_specs=[
            pl.BlockSpec((1, BLK, D), lambda s, b: (b, s, 0)),
            pl.BlockSpec((BLK, D), lambda s, b: (s, 0)),
        ],
        out_specs=pl.BlockSpec((1, BLK, D), lambda s, b: (b, s, 0)),
        out_shape=jax.ShapeDtypeStruct((B, S, D), x.dtype),
    )(x, position_embeddings[:S])


# speedup vs baseline: 2.1246x; 1.1049x over previous
import math

import jax
import jax.numpy as jnp
from jax.experimental import pallas as pl


def _pe_add_kernel(x_ref, pe_ref, o_ref, *, scale):
    o_ref[...] = x_ref[...] * scale + pe_ref[...]


def kernel(x, position_embeddings):
    B, S, D = x.shape
    scale = math.sqrt(D)
    BLK = 1024
    grid = (S // BLK, B)
    import functools

    return pl.pallas_call(
        functools.partial(_pe_add_kernel, scale=scale),
        grid=grid,
        in_specs=[
            pl.BlockSpec((1, BLK, D), lambda s, b: (b, s, 0)),
            pl.BlockSpec((BLK, D), lambda s, b: (s, 0)),
        ],
        out_specs=pl.BlockSpec((1, BLK, D), lambda s, b: (b, s, 0)),
        out_shape=jax.ShapeDtypeStruct((B, S, D), x.dtype),
    )(x, position_embeddings[:S])


# speedup vs baseline: 2.3024x; 1.0837x over previous
import math

import jax
import jax.numpy as jnp
from jax.experimental import pallas as pl


def _pe_add_kernel(x_ref, pe_ref, o_ref, *, scale):
    o_ref[...] = x_ref[...] * scale + pe_ref[...]


def kernel(x, position_embeddings):
    B, S, D = x.shape
    scale = math.sqrt(D)
    BLK = 2048
    grid = (S // BLK, B)
    import functools

    return pl.pallas_call(
        functools.partial(_pe_add_kernel, scale=scale),
        grid=grid,
        in_specs=[
            pl.BlockSpec((1, BLK, D), lambda s, b: (b, s, 0)),
            pl.BlockSpec((BLK, D), lambda s, b: (s, 0)),
        ],
        out_specs=pl.BlockSpec((1, BLK, D), lambda s, b: (b, s, 0)),
        out_shape=jax.ShapeDtypeStruct((B, S, D), x.dtype),
    )(x, position_embeddings[:S])
